# Initial kernel scaffold; baseline (speedup 1.0000x reference)
import jax
import jax.numpy as jnp
from jax.experimental import pallas as pl


def kernel(vertex_normals, faces):
    raise NotImplementedError("write your pallas kernel here")


# speedup vs baseline: 34.9987x; 34.9987x over previous
import functools

import jax
import jax.numpy as jnp
from jax import lax
from jax.experimental import pallas as pl
from jax.experimental.pallas import tpu as pltpu
from jax.experimental.pallas import tpu_sc as plsc

B = 8
V = 100000
F = 200000

VP = 102400
NC, NS = 2, 16
NW = NC * NS
FPT = 6400
FP = FPT * NW
CH = 640
NCHUNK = FPT // CH
GA = 128
NGA = CH // GA


def _products_body(x_ref, y_ref, z_ref, o_ref):
    x = x_ref[...]
    y = y_ref[...]
    z = z_ref[...]
    o_ref[0, :, :] = x * y
    o_ref[1, :, :] = x * z
    o_ref[2, :, :] = z * y


def _products(x, y, z):
    blk = 2048
    grid = VP // blk
    return pl.pallas_call(
        _products_body,
        grid=(grid,),
        in_specs=[pl.BlockSpec((B, blk), lambda i: (0, i))] * 3,
        out_specs=pl.BlockSpec((3, B, blk), lambda i: (0, 0, i)),
        out_shape=jax.ShapeDtypeStruct((3, B, VP), jnp.float32),
    )(x, y, z)


def _gather_sum(table, faces1):
    mesh = plsc.VectorSubcoreMesh(core_axis_name="c", subcore_axis_name="s")

    @functools.partial(
        pl.kernel,
        mesh=mesh,
        compiler_params=pltpu.CompilerParams(use_tc_tiling_on_sc=False),
        out_type=jax.ShapeDtypeStruct((FP, 24), jnp.float32),
        scratch_types=[
            pltpu.VMEM((3 * CH,), jnp.int32),
            pltpu.VMEM((CH, 24), jnp.float32),
            pltpu.VMEM((CH, 24), jnp.float32),
            pltpu.VMEM((CH, 24), jnp.float32),
            pltpu.VMEM((CH, 24), jnp.float32),
            pltpu.SemaphoreType.DMA,
        ],
    )
    def k(table_hbm, faces_hbm, out_hbm, idx_s, r0, r1, r2, o_s, sem):
        wid = lax.axis_index("s") * NC + lax.axis_index("c")
        fbase0 = wid * FPT
        rbufs = (r0, r1, r2)
        for kk in range(NCHUNK):
            pltpu.sync_copy(
                faces_hbm.at[pl.ds((wid * NCHUNK + kk) * 3 * CH, 3 * CH)],
                idx_s,
            )
            copies = []
            for t in range(3):
                for j in range(NGA):
                    copies.append(
                        pltpu.async_copy(
                            table_hbm.at[idx_s.at[pl.ds(t * CH + j * GA, GA)]],
                            rbufs[t].at[pl.ds(j * GA, GA), :],
                            sem,
                        )
                    )
            for c in copies:
                c.wait()

            def body(i, carry):
                for h in (0, 8):
                    sl = pl.ds(h, 16)
                    o_s[i, sl] = 1.0 - (r0[i, sl] + r1[i, sl] + r2[i, sl])
                return carry

            lax.fori_loop(0, CH, body, 0)
            pltpu.sync_copy(o_s, out_hbm.at[pl.ds(fbase0 + kk * CH, CH), :])

    return k(table, faces1)


def kernel(vertex_normals, faces):
    faces = jnp.squeeze(faces)
    x = jnp.pad(vertex_normals[:, :, 0], ((0, 0), (0, VP - V)))
    y = jnp.pad(vertex_normals[:, :, 1], ((0, 0), (0, VP - V)))
    z = jnp.pad(vertex_normals[:, :, 2], ((0, 0), (0, VP - V)))
    prods = _products(x, y, z)
    table = prods.transpose(2, 0, 1).reshape(VP, 3 * B)
    facesT = jnp.pad(faces.T, ((0, 0), (0, FP - F)))
    faces1 = facesT.reshape(3, NW, NCHUNK, CH).transpose(1, 2, 0, 3).reshape(-1)
    out24 = _gather_sum(table, faces1)
    return out24[:F].reshape(F, 3, B).transpose(1, 2, 0)
